# baseline (speedup 1.0000x reference)
import jax
import jax.numpy as jnp
from jax.experimental import pallas as pl
from jax.experimental.pallas import tpu as pltpu

N = 10000
D = 128
H = 128
G = 128
HB = 256

BM = 400

_INV_N = 1.0 / N


def _pool_body(adj_ref, hb_ref, out_ref):
    def step(k, acc):
        off = pl.multiple_of(k * 256, 256)
        return acc + jax.lax.dot_general(
            adj_ref[:, pl.ds(off, 256)], hb_ref[pl.ds(off, 256), :],
            (((1,), (0,)), ((), ())), preferred_element_type=jnp.float32)
    acc = jax.lax.fori_loop(0, N // 256, step,
                            jnp.zeros((BM, HB), jnp.float32))
    rem = (N // 256) * 256
    out_ref[...] = acc + jax.lax.dot_general(
        adj_ref[:, rem:], hb_ref[rem:, :], (((1,), (0,)), ((), ())),
        preferred_element_type=jnp.float32)


def _pool(adj, hb):
    return pl.pallas_call(
        _pool_body,
        grid=(N // BM,),
        in_specs=[
            pl.BlockSpec((BM, N), lambda i: (i, 0)),
            pl.BlockSpec((N, HB), lambda i: (0, 0)),
        ],
        out_specs=pl.BlockSpec((BM, HB), lambda i: (i, 0)),
        out_shape=jax.ShapeDtypeStruct((N, HB), jnp.float32),
        compiler_params=pltpu.CompilerParams(
            dimension_semantics=("parallel",)),
    )(adj, hb)


def _dot(a, b):
    return jax.lax.dot_general(
        a, b, (((1,), (0,)), ((), ())), preferred_element_type=jnp.float32)


def _bf16(u):
    b = jax.lax.bitcast_convert_type(u, jnp.uint32)
    r = b + jnp.uint32(0x7FFF) + ((b >> 16) & jnp.uint32(1))
    return jax.lax.bitcast_convert_type(r & jnp.uint32(0xFFFF0000), jnp.float32)


def _mlp_bn(p_full, t):
    pooled = _bf16(p_full[:, :H] / p_full[:, H:H + 1])
    relu = lambda u: jnp.maximum(u, 0.0)
    t1 = _bf16(relu(_dot(pooled, t[0]) + t[1]))
    z = _dot(t1, t[2]) + t[3]
    m = jnp.sum(z, axis=0, keepdims=True) * _INV_N
    v = jnp.sum((z - m) ** 2, axis=0, keepdims=True) * _INV_N
    return relu((z - m) / jnp.sqrt(v + 1e-5) * t[4] + t[5])


def _update_body(p_ref, *rest):
    wrefs, out_ref = rest[:6], rest[6]
    out_ref[...] = _mlp_bn(p_ref[...], [w[...] for w in wrefs])


def _update(p_full, weights):
    return pl.pallas_call(
        _update_body,
        out_shape=jax.ShapeDtypeStruct((N, H), jnp.float32),
    )(p_full, *weights)


def _update_pool_body(p_ref, gp_ref, *rest):
    wrefs, h_ref, out_ref = rest[:6], rest[6], rest[7]
    h = _mlp_bn(p_ref[...], [w[...] for w in wrefs])
    h_ref[...] = h
    out_ref[...] = _dot(gp_ref[...], h)


def _update_pool(p_full, graph_pool, weights):
    return pl.pallas_call(
        _update_pool_body,
        out_shape=(
            jax.ShapeDtypeStruct((N, H), jnp.float32),
            jax.ShapeDtypeStruct((G, H), jnp.float32),
        ),
    )(p_full, graph_pool, *weights)


def _augment(h):
    pad = jnp.zeros((N, HB - H - 1), jnp.float32)
    return jnp.concatenate([h, jnp.ones((N, 1), jnp.float32), pad], axis=1)


def kernel(x, graph_pool, padded_nei, adj, W0_1, b0_1, W0_2, b0_2, g0, be0,
           W1_1, b1_1, W1_2, b1_2, g1, be1):
    w0 = (W0_1, b0_1.reshape(1, H), W0_2, b0_2.reshape(1, H),
          g0.reshape(1, H), be0.reshape(1, H))
    w1 = (W1_1, b1_1.reshape(1, H), W1_2, b1_2.reshape(1, H),
          g1.reshape(1, H), be1.reshape(1, H))
    h1 = _update(_pool(adj, _augment(x)), w0)
    h2, pooled_h = _update_pool(_pool(adj, _augment(h1)), graph_pool, w1)
    return (pooled_h, h2)
